# Initial kernel scaffold; baseline (speedup 1.0000x reference)
import jax
import jax.numpy as jnp
from jax.experimental import pallas as pl


def kernel(x, edge_index, edge_attr, batch, W1_rel, b1, W1_root, W2_rel, b2, W2_root, Wl, bl):
    raise NotImplementedError("write your pallas kernel here")


# speedup vs baseline: 11.2285x; 11.2285x over previous
import functools

import jax
import jax.numpy as jnp
from jax import lax
from jax.experimental import pallas as pl
from jax.experimental.pallas import tpu as pltpu
from jax.experimental.pallas import tpu_sc as plsc

_N = 10000
_E = 320000
_DH = 16
_G = 128
_NC = 2
_NS = 16
_NW = _NC * _NS
_CH = 128
_NCHUNK = 80
_EPT = _CH * _NCHUNK
_EPAD = _EPT * _NW
_NPAD = 10240
_ZPT = _NPAD // _NS
_OPT = _N // _NS


def _mm1_body(x_ref, wr_ref, wo_ref, b_ref, y_ref, r_ref):
    x = x_ref[...]
    y_ref[...] = jnp.dot(x, wr_ref[...], preferred_element_type=jnp.float32, precision=lax.Precision.HIGHEST)
    r_ref[...] = (
        jnp.dot(x, wo_ref[...], preferred_element_type=jnp.float32, precision=lax.Precision.HIGHEST) + b_ref[...]
    )


def _mid_body(a_ref, r_ref, wr_ref, wo_ref, b_ref, y2_ref, r2_ref):
    h = jnp.maximum(a_ref[0, :_N] + a_ref[1, :_N] + r_ref[...], 0.0)
    y2_ref[...] = jnp.dot(h, wr_ref[...], preferred_element_type=jnp.float32, precision=lax.Precision.HIGHEST)
    r2_ref[...] = (
        jnp.dot(h, wo_ref[...], preferred_element_type=jnp.float32, precision=lax.Precision.HIGHEST) + b_ref[...]
    )


def _final_body(a_ref, r_ref, batch_ref, wl_ref, bl_ref, out_ref):
    h2 = a_ref[0, :_N] + a_ref[1, :_N] + r_ref[...]
    z = jnp.dot(h2, wl_ref[...], preferred_element_type=jnp.float32, precision=lax.Precision.HIGHEST)
    gid = lax.broadcasted_iota(jnp.int32, (1, _G), 1)
    onehot = (batch_ref[...] == gid).astype(jnp.float32)
    zz = jnp.concatenate([z, jnp.ones_like(z)], axis=1)
    sc = lax.dot_general(
        onehot, zz, (((0,), (0,)), ((), ())),
        preferred_element_type=jnp.float32,
        precision=lax.Precision.HIGHEST,
    )
    out_ref[...] = sc[:, 0:1] / jnp.maximum(sc[:, 1:2], 1.0) + bl_ref[...]


def _segsum_body(y_hbm, src_hbm, dst_hbm, z_hbm, out_hbm,
                 src_v, dst_v, rows_v, agg, sem):
    c = lax.axis_index("c")
    s = lax.axis_index("s")
    wid = c * _NS + s

    pltpu.sync_copy(z_hbm, agg.at[pl.ds(s * _ZPT, _ZPT)])
    pltpu.sync_copy(src_hbm.at[wid], src_v)
    pltpu.sync_copy(dst_hbm.at[wid], dst_v)
    plsc.subcore_barrier()

    def body(j, carry):
        pltpu.async_copy(y_hbm.at[src_v.at[j]], rows_v, sem).wait()
        pltpu.sync_copy(rows_v, agg.at[dst_v.at[j]], add=True)
        return carry

    lax.fori_loop(0, _NCHUNK, body, 0)
    plsc.subcore_barrier()
    pltpu.sync_copy(
        agg.at[pl.ds(s * _ZPT, _ZPT)],
        out_hbm.at[c, pl.ds(s * _ZPT, _ZPT)],
    )


def _segsum(y, srcp, dstp, zblk):
    mesh = plsc.VectorSubcoreMesh(core_axis_name="c", subcore_axis_name="s")
    f = functools.partial(
        pl.kernel,
        out_type=jax.ShapeDtypeStruct((_NC, _NPAD, _DH), jnp.float32),
        mesh=mesh,
        scratch_types=[
            pltpu.VMEM((_NCHUNK, _CH), jnp.int32),
            pltpu.VMEM((_NCHUNK, _CH), jnp.int32),
            pltpu.VMEM((_CH, _DH), jnp.float32),
            pltpu.VMEM_SHARED((_NPAD, _DH), jnp.float32),
            pltpu.SemaphoreType.DMA,
        ],
        compiler_params=pltpu.CompilerParams(use_tc_tiling_on_sc=False),
    )(_segsum_body)
    return f(y, srcp, dstp, zblk)


def kernel(x, edge_index, edge_attr, batch,
           W1_rel, b1, W1_root, W2_rel, b2, W2_root, Wl, bl):
    del edge_attr
    src = edge_index[0]
    dst = edge_index[1]
    pad = _EPAD - _E
    srcp = jnp.concatenate(
        [src, jnp.zeros((pad,), jnp.int32)]).reshape(_NW, _NCHUNK, _CH)
    dstp = jnp.concatenate(
        [dst, jnp.full((pad,), _N, jnp.int32)]).reshape(_NW, _NCHUNK, _CH)
    zblk = jnp.zeros((_ZPT, _DH), jnp.float32)

    out1 = jax.ShapeDtypeStruct((_N, _DH), jnp.float32)
    y1, r1b = pl.pallas_call(
        _mm1_body, out_shape=(out1, out1),
    )(x, W1_rel, W1_root, b1.reshape(1, _DH))

    a1 = _segsum(y1, srcp, dstp, zblk)

    y2, r2b = pl.pallas_call(
        _mid_body, out_shape=(out1, out1),
    )(a1, r1b, W2_rel, W2_root, b2.reshape(1, _DH))

    a2 = _segsum(y2, srcp, dstp, zblk)

    out2d = pl.pallas_call(
        _final_body, out_shape=jax.ShapeDtypeStruct((_G, 1), jnp.float32),
    )(a2, r2b, batch.reshape(_N, 1), Wl, bl.reshape(1, 1))
    return out2d[:, 0]


# speedup vs baseline: 14.0299x; 1.2495x over previous
import functools

import jax
import jax.numpy as jnp
from jax import lax
from jax.experimental import pallas as pl
from jax.experimental.pallas import tpu as pltpu
from jax.experimental.pallas import tpu_sc as plsc

_N = 10000
_E = 320000
_DH = 16
_G = 128
_NC = 2
_NS = 16
_NW = _NC * _NS
_CE = 1024
_NB = 10
_EPT = _CE * _NB
_EPAD = _EPT * _NW
_NPAD = 10240
_ZPT = _NPAD // _NS
_OPT = _N // _NS


def _mm1_body(x_ref, wr_ref, wo_ref, b_ref, y_ref, r_ref):
    x = x_ref[...]
    y_ref[...] = jnp.dot(x, wr_ref[...], preferred_element_type=jnp.float32, precision=lax.Precision.HIGHEST)
    r_ref[...] = (
        jnp.dot(x, wo_ref[...], preferred_element_type=jnp.float32, precision=lax.Precision.HIGHEST) + b_ref[...]
    )


def _mid_body(a_ref, r_ref, wr_ref, wo_ref, b_ref, y2_ref, r2_ref):
    h = jnp.maximum(a_ref[0, :_N] + a_ref[1, :_N] + r_ref[...], 0.0)
    y2_ref[...] = jnp.dot(h, wr_ref[...], preferred_element_type=jnp.float32, precision=lax.Precision.HIGHEST)
    r2_ref[...] = (
        jnp.dot(h, wo_ref[...], preferred_element_type=jnp.float32, precision=lax.Precision.HIGHEST) + b_ref[...]
    )


def _final_body(a_ref, r_ref, batch_ref, wl_ref, bl_ref, out_ref):
    h2 = a_ref[0, :_N] + a_ref[1, :_N] + r_ref[...]
    z = jnp.dot(h2, wl_ref[...], preferred_element_type=jnp.float32, precision=lax.Precision.HIGHEST)
    gid = lax.broadcasted_iota(jnp.int32, (1, _G), 1)
    onehot = (batch_ref[...] == gid).astype(jnp.float32)
    zz = jnp.concatenate([z, jnp.ones_like(z)], axis=1)
    sc = lax.dot_general(
        onehot, zz, (((0,), (0,)), ((), ())),
        preferred_element_type=jnp.float32,
        precision=lax.Precision.HIGHEST,
    )
    out_ref[...] = sc[:, 0:1] / jnp.maximum(sc[:, 1:2], 1.0) + bl_ref[...]


def _segsum_body(y_hbm, src_hbm, dst_hbm, z_hbm, out_hbm,
                 *rest):
    srcbs = rest[:_NB]
    dstbs = rest[_NB:2 * _NB]
    rows_v, rows_v2, agg, sem, sem2 = rest[2 * _NB:]
    c = lax.axis_index("c")
    s = lax.axis_index("s")
    wid = c * _NS + s

    pltpu.sync_copy(z_hbm, agg.at[pl.ds(s * _ZPT, _ZPT)])
    for j in range(_NB):
        pltpu.sync_copy(src_hbm.at[wid, j], srcbs[j])
        pltpu.sync_copy(dst_hbm.at[wid, j], dstbs[j])
    plsc.subcore_barrier()

    bufs = (rows_v, rows_v2)
    sems = (sem, sem2)
    g = [None] * _NB
    g[0] = pltpu.async_copy(y_hbm.at[srcbs[0]], bufs[0], sems[0])
    for j in range(_NB):
        if j + 1 < _NB:
            g[j + 1] = pltpu.async_copy(
                y_hbm.at[srcbs[j + 1]],
                bufs[(j + 1) % 2], sems[(j + 1) % 2])
        g[j].wait()
        pltpu.sync_copy(bufs[j % 2], agg.at[dstbs[j]], add=True)
    plsc.subcore_barrier()
    pltpu.sync_copy(
        agg.at[pl.ds(s * _ZPT, _ZPT)],
        out_hbm.at[c, pl.ds(s * _ZPT, _ZPT)],
    )


def _segsum(y, srcp, dstp, zblk):
    mesh = plsc.VectorSubcoreMesh(core_axis_name="c", subcore_axis_name="s")
    f = functools.partial(
        pl.kernel,
        out_type=jax.ShapeDtypeStruct((_NC, _NPAD, _DH), jnp.float32),
        mesh=mesh,
        scratch_types=[
            *[pltpu.VMEM((_CE,), jnp.int32) for _ in range(2 * _NB)],
            pltpu.VMEM((_CE, _DH), jnp.float32),
            pltpu.VMEM((_CE, _DH), jnp.float32),
            pltpu.VMEM_SHARED((_NPAD, _DH), jnp.float32),
            pltpu.SemaphoreType.DMA,
            pltpu.SemaphoreType.DMA,
        ],
        compiler_params=pltpu.CompilerParams(use_tc_tiling_on_sc=False),
    )(_segsum_body)
    return f(y, srcp, dstp, zblk)


def kernel(x, edge_index, edge_attr, batch,
           W1_rel, b1, W1_root, W2_rel, b2, W2_root, Wl, bl):
    del edge_attr
    src = edge_index[0]
    dst = edge_index[1]
    pad = _EPAD - _E
    srcp = jnp.concatenate(
        [src, jnp.zeros((pad,), jnp.int32)]).reshape(_NW, _NB, _CE)
    dstp = jnp.concatenate(
        [dst, jnp.full((pad,), _N, jnp.int32)]).reshape(_NW, _NB, _CE)
    zblk = jnp.zeros((_ZPT, _DH), jnp.float32)

    out1 = jax.ShapeDtypeStruct((_N, _DH), jnp.float32)
    y1, r1b = pl.pallas_call(
        _mm1_body, out_shape=(out1, out1),
    )(x, W1_rel, W1_root, b1.reshape(1, _DH))

    a1 = _segsum(y1, srcp, dstp, zblk)

    y2, r2b = pl.pallas_call(
        _mid_body, out_shape=(out1, out1),
    )(a1, r1b, W2_rel, W2_root, b2.reshape(1, _DH))

    a2 = _segsum(y2, srcp, dstp, zblk)

    out2d = pl.pallas_call(
        _final_body, out_shape=jax.ShapeDtypeStruct((_G, 1), jnp.float32),
    )(a2, r2b, batch.reshape(_N, 1), Wl, bl.reshape(1, 1))
    return out2d[:, 0]


# speedup vs baseline: 19.0282x; 1.3563x over previous
import functools

import jax
import jax.numpy as jnp
from jax import lax
from jax.experimental import pallas as pl
from jax.experimental.pallas import tpu as pltpu
from jax.experimental.pallas import tpu_sc as plsc

_N = 10000
_E = 320000
_DH = 16
_G = 128
_NC = 2
_NS = 16
_NW = _NC * _NS
_CE = 1000
_NB = 10
_EPT = _CE * _NB
_NPAD = 10240
_ZPT = _NPAD // _NS
_OPT = _N // _NS


def _mm1_body(x_ref, wr_ref, wo_ref, b_ref, y_ref, r_ref):
    x = x_ref[...]
    y_ref[...] = jnp.dot(x, wr_ref[...], preferred_element_type=jnp.float32, precision=lax.Precision.HIGHEST)
    r_ref[...] = (
        jnp.dot(x, wo_ref[...], preferred_element_type=jnp.float32, precision=lax.Precision.HIGHEST) + b_ref[...]
    )


def _mid_body(a_ref, r_ref, wr_ref, wo_ref, b_ref, y2_ref, r2_ref):
    h = jnp.maximum(a_ref[0, :_N] + a_ref[1, :_N] + r_ref[...], 0.0)
    y2_ref[...] = jnp.dot(h, wr_ref[...], preferred_element_type=jnp.float32, precision=lax.Precision.HIGHEST)
    r2_ref[...] = (
        jnp.dot(h, wo_ref[...], preferred_element_type=jnp.float32, precision=lax.Precision.HIGHEST) + b_ref[...]
    )


def _final_body(a_ref, r_ref, batch_ref, wl_ref, bl_ref, out_ref):
    h2 = a_ref[0, :_N] + a_ref[1, :_N] + r_ref[...]
    z = jnp.dot(h2, wl_ref[...], preferred_element_type=jnp.float32, precision=lax.Precision.HIGHEST)
    gid = lax.broadcasted_iota(jnp.int32, (1, _G), 1)
    onehot = (batch_ref[...] == gid).astype(jnp.float32)
    zz = jnp.concatenate([z, jnp.ones_like(z)], axis=1)
    sc = lax.dot_general(
        onehot, zz, (((0,), (0,)), ((), ())),
        preferred_element_type=jnp.float32,
        precision=lax.Precision.HIGHEST,
    )
    out_ref[...] = sc[:, 0:1] / jnp.maximum(sc[:, 1:2], 1.0) + bl_ref[...]


def _segsum_body(y_hbm, src_hbm, dst_hbm, z_hbm, out_hbm,
                 *rest):
    srcbs = rest[:_NB]
    dstbs = rest[_NB:2 * _NB]
    rows_v, rows_v2, agg, sem, sem2 = rest[2 * _NB:]
    c = lax.axis_index("c")
    s = lax.axis_index("s")
    wid = c * _NS + s

    pltpu.sync_copy(z_hbm, agg.at[pl.ds(s * _ZPT, _ZPT)])
    for j in range(_NB):
        pltpu.sync_copy(src_hbm.at[wid, j], srcbs[j])
        pltpu.sync_copy(dst_hbm.at[wid, j], dstbs[j])
    plsc.subcore_barrier()

    bufs = (rows_v, rows_v2)
    sems = (sem, sem2)
    g = [None] * _NB
    g[0] = pltpu.async_copy(y_hbm.at[srcbs[0]], bufs[0], sems[0])
    for j in range(_NB):
        if j + 1 < _NB:
            g[j + 1] = pltpu.async_copy(
                y_hbm.at[srcbs[j + 1]],
                bufs[(j + 1) % 2], sems[(j + 1) % 2])
        g[j].wait()
        pltpu.sync_copy(bufs[j % 2], agg.at[dstbs[j]], add=True)
    plsc.subcore_barrier()
    pltpu.sync_copy(
        agg.at[pl.ds(s * _ZPT, _ZPT)],
        out_hbm.at[c, pl.ds(s * _ZPT, _ZPT)],
    )


def _segsum(y, srcp, dstp, zblk):
    mesh = plsc.VectorSubcoreMesh(core_axis_name="c", subcore_axis_name="s")
    f = functools.partial(
        pl.kernel,
        out_type=jax.ShapeDtypeStruct((_NC, _NPAD, _DH), jnp.float32),
        mesh=mesh,
        scratch_types=[
            *[pltpu.VMEM((_CE,), jnp.int32) for _ in range(2 * _NB)],
            pltpu.VMEM((_CE, _DH), jnp.float32),
            pltpu.VMEM((_CE, _DH), jnp.float32),
            pltpu.VMEM_SHARED((_NPAD, _DH), jnp.float32),
            pltpu.SemaphoreType.DMA,
            pltpu.SemaphoreType.DMA,
        ],
        compiler_params=pltpu.CompilerParams(use_tc_tiling_on_sc=False),
    )(_segsum_body)
    return f(y, srcp, dstp, zblk)


def kernel(x, edge_index, edge_attr, batch,
           W1_rel, b1, W1_root, W2_rel, b2, W2_root, Wl, bl):
    del edge_attr
    srcp = edge_index[0].reshape(_NW, _NB, _CE)
    dstp = edge_index[1].reshape(_NW, _NB, _CE)
    zblk = jnp.zeros((_ZPT, _DH), jnp.float32)

    out1 = jax.ShapeDtypeStruct((_N, _DH), jnp.float32)
    y1, r1b = pl.pallas_call(
        _mm1_body, out_shape=(out1, out1),
    )(x, W1_rel, W1_root, b1.reshape(1, _DH))

    a1 = _segsum(y1, srcp, dstp, zblk)

    y2, r2b = pl.pallas_call(
        _mid_body, out_shape=(out1, out1),
    )(a1, r1b, W2_rel, W2_root, b2.reshape(1, _DH))

    a2 = _segsum(y2, srcp, dstp, zblk)

    out2d = pl.pallas_call(
        _final_body, out_shape=jax.ShapeDtypeStruct((_G, 1), jnp.float32),
    )(a2, r2b, batch.reshape(_N, 1), Wl, bl.reshape(1, 1))
    return out2d[:, 0]
